# Initial kernel scaffold; baseline (speedup 1.0000x reference)
import jax
import jax.numpy as jnp
from jax.experimental import pallas as pl


def kernel(h, edge_index, edge_attr, W1, b1, W2, b2, W3, b3, W4, b4):
    raise NotImplementedError("write your pallas kernel here")


# speedup vs baseline: 3.1199x; 3.1199x over previous
import jax
import jax.numpy as jnp
from jax import lax
from jax.experimental import pallas as pl
from jax.experimental.pallas import tpu as pltpu
from jax.experimental.pallas import tpu_sc as plsc

N_NODES = 10000
N_EDGES = 320000
HID = 128
NC, NS, L = 2, 16, 16
NW = NC * NS
C = 128
NR = N_EDGES // C
AGGW = HID + L
RPT = N_NODES // NS


def _pre_body(h_ref, w1a_ref, w1b_ref, b1_ref, p_ref, q_ref):
    h = h_ref[...]
    p_ref[...] = jnp.dot(h, w1a_ref[...], preferred_element_type=jnp.float32) + b1_ref[...]
    q_ref[...] = jnp.dot(h, w1b_ref[...], preferred_element_type=jnp.float32)


def _post_body(h_ref, p0_ref, p1_ref, d0_ref, d1_ref, w2_ref, b2_ref,
               w3a_ref, w3b_ref, b3_ref, w4_ref, b4_ref, o_ref):
    aggh = p0_ref[...] + p1_ref[...]
    deg = d0_ref[:, :1] + d1_ref[:, :1]
    agg = jnp.dot(aggh, w2_ref[...], preferred_element_type=jnp.float32) + deg * b2_ref[...]
    h = h_ref[...]
    hupd = jnp.maximum(
        jnp.dot(h, w3a_ref[...], preferred_element_type=jnp.float32)
        + jnp.dot(agg, w3b_ref[...], preferred_element_type=jnp.float32)
        + b3_ref[...], 0.0)
    o_ref[...] = h + jnp.dot(hupd, w4_ref[...], preferred_element_type=jnp.float32) + b4_ref[...]


def _sc_body(p_hbm, q_hbm, src_hbm, dst_hbm, ea_hbm, w1c_hbm,
             agg_out, deg_out,
             idx_v, ea_v, pbuf, qbuf, ones_v, wc_v, agg_sh, deg_sh,
             sem_p, sem_q):
    c = lax.axis_index("c")
    s = lax.axis_index("s")
    w = s * NC + c

    zero16 = jnp.zeros((L,), jnp.float32)

    def zbody(e, carry):
        for k in range(HID // L):
            pbuf[e, pl.ds(k * L, L)] = zero16
        ones_v[e, :] = zero16
        return carry

    lax.fori_loop(0, C, zbody, 0)
    for j in range(RPT // 125):
        pltpu.sync_copy(pbuf.at[pl.ds(0, 125)],
                        agg_sh.at[pl.ds(s * RPT + j * 125, 125)])
        pltpu.sync_copy(ones_v.at[pl.ds(0, 125)],
                        deg_sh.at[pl.ds(s * RPT + j * 125, 125)])
    plsc.subcore_barrier()

    lanes = lax.broadcasted_iota(jnp.int32, (L,), 0)
    onespat = jnp.where(lanes == 0, 1.0, 0.0).astype(jnp.float32)

    def obody(e, carry):
        ones_v[e, :] = onespat
        return carry

    lax.fori_loop(0, C, obody, 0)

    pltpu.sync_copy(w1c_hbm, wc_v)
    wc = [wc_v[pl.ds(k * L, L)] for k in range(HID // L)]

    n_rows = 78 + jnp.where(w < NR - 78 * NW, 1, 0)

    def row_body(i, carry):
        r = w + i * NW
        pltpu.sync_copy(src_hbm.at[r], idx_v.at[0])
        pltpu.sync_copy(dst_hbm.at[r], idx_v.at[1])
        pltpu.sync_copy(ea_hbm.at[r], ea_v)
        cp_p = pltpu.async_copy(p_hbm.at[idx_v.at[0]], pbuf, sem_p)
        cp_q = pltpu.async_copy(q_hbm.at[idx_v.at[1]], qbuf, sem_q)
        cp_p.wait()
        cp_q.wait()

        def edge_body(e, ecarry):
            ea_b = plsc.load_gather(ea_v, [jnp.full((L,), e, jnp.int32)])
            for k in range(HID // L):
                pv = pbuf[e, pl.ds(k * L, L)]
                qv = qbuf[e, pl.ds(k * L, L)]
                pbuf[e, pl.ds(k * L, L)] = jnp.maximum(pv + qv + ea_b * wc[k], 0.0)
            return ecarry

        lax.fori_loop(0, C, edge_body, 0)
        pltpu.sync_copy(pbuf, agg_sh.at[idx_v.at[1]], add=True)
        pltpu.sync_copy(ones_v, deg_sh.at[idx_v.at[1]], add=True)
        return carry

    lax.fori_loop(0, n_rows, row_body, 0)

    plsc.subcore_barrier()
    pltpu.sync_copy(agg_sh.at[pl.ds(s * RPT, RPT)],
                    agg_out.at[c, pl.ds(s * RPT, RPT)])
    pltpu.sync_copy(deg_sh.at[pl.ds(s * RPT, RPT)],
                    deg_out.at[c, pl.ds(s * RPT, RPT)])


def kernel(h, edge_index, edge_attr, W1, b1, W2, b2, W3, b3, W4, b4):
    ei = edge_index.astype(jnp.int32)
    src2 = ei[0].reshape(NR, C)
    dst2 = ei[1].reshape(NR, C)
    ea2 = edge_attr.astype(jnp.float32).reshape(NR, C)
    W1aT = W1[:, :HID].T
    W1bT = W1[:, HID:2 * HID].T
    w1c = W1[:, 2 * HID]

    P, Q = pl.pallas_call(
        _pre_body,
        grid=(10,),
        in_specs=[
            pl.BlockSpec((1000, HID), lambda i: (i, 0)),
            pl.BlockSpec((HID, HID), lambda i: (0, 0)),
            pl.BlockSpec((HID, HID), lambda i: (0, 0)),
            pl.BlockSpec((1, HID), lambda i: (0, 0)),
        ],
        out_specs=[pl.BlockSpec((1000, HID), lambda i: (i, 0))] * 2,
        out_shape=[jax.ShapeDtypeStruct((N_NODES, HID), jnp.float32)] * 2,
    )(h, W1aT, W1bT, b1.reshape(1, HID))

    mesh = plsc.VectorSubcoreMesh(core_axis_name="c", subcore_axis_name="s",
                                  num_cores=NC, num_subcores=NS)
    agg_parts, deg_parts = pl.kernel(
        _sc_body,
        out_type=(jax.ShapeDtypeStruct((NC, N_NODES, HID), jnp.float32),
                  jax.ShapeDtypeStruct((NC, N_NODES, L), jnp.float32)),
        mesh=mesh,
        scratch_types=[
            pltpu.VMEM((2, C), jnp.int32),
            pltpu.VMEM((C,), jnp.float32),
            pltpu.VMEM((C, HID), jnp.float32),
            pltpu.VMEM((C, HID), jnp.float32),
            pltpu.VMEM((C, L), jnp.float32),
            pltpu.VMEM((HID,), jnp.float32),
            pltpu.VMEM_SHARED((N_NODES, HID), jnp.float32),
            pltpu.VMEM_SHARED((N_NODES, L), jnp.float32),
            pltpu.SemaphoreType.DMA,
            pltpu.SemaphoreType.DMA,
        ],
        compiler_params=pltpu.CompilerParams(use_tc_tiling_on_sc=False,
                                             needs_layout_passes=False),
    )(P, Q, src2, dst2, ea2, w1c)

    out = pl.pallas_call(
        _post_body,
        grid=(10,),
        in_specs=[
            pl.BlockSpec((1000, HID), lambda i: (i, 0)),
            pl.BlockSpec((1000, HID), lambda i: (i, 0)),
            pl.BlockSpec((1000, HID), lambda i: (i, 0)),
            pl.BlockSpec((1000, L), lambda i: (i, 0)),
            pl.BlockSpec((1000, L), lambda i: (i, 0)),
            pl.BlockSpec((HID, HID), lambda i: (0, 0)),
            pl.BlockSpec((1, HID), lambda i: (0, 0)),
            pl.BlockSpec((HID, HID), lambda i: (0, 0)),
            pl.BlockSpec((HID, HID), lambda i: (0, 0)),
            pl.BlockSpec((1, HID), lambda i: (0, 0)),
            pl.BlockSpec((HID, HID), lambda i: (0, 0)),
            pl.BlockSpec((1, HID), lambda i: (0, 0)),
        ],
        out_specs=pl.BlockSpec((1000, HID), lambda i: (i, 0)),
        out_shape=jax.ShapeDtypeStruct((N_NODES, HID), jnp.float32),
    )(h, agg_parts[0], agg_parts[1], deg_parts[0], deg_parts[1],
      W2.T, b2.reshape(1, HID),
      W3[:, :HID].T, W3[:, HID:].T, b3.reshape(1, HID),
      W4.T, b4.reshape(1, HID))
    return out
